# Initial kernel scaffold; baseline (speedup 1.0000x reference)
import jax
import jax.numpy as jnp
from jax.experimental import pallas as pl


def kernel(emb, bases0, coeffs0, self_loop0, bases1, coeffs1, self_loop1, edge_index, edge_type):
    raise NotImplementedError("write your pallas kernel here")


# speedup vs baseline: 9.1353x; 9.1353x over previous
import functools

import jax
import jax.numpy as jnp
from jax import lax
from jax.experimental import pallas as pl
from jax.experimental.pallas import tpu as pltpu
from jax.experimental.pallas import tpu_sc as plsc

N = 50000
E = 800000
D = 64
R = 4
B = 30
H = D // 2

NC = 2
NS = 16
CHUNK = 128
RPT = 3128
N_PAD = RPT * NS

EPT = E // NS
NFULL = EPT // CHUNK
TAIL = EPT - NFULL * CHUNK

DEPT = (E // NC) // NS
DNFULL = DEPT // CHUNK
DTAIL = DEPT - DNFULL * CHUNK

_MESH = plsc.VectorSubcoreMesh(core_axis_name="c", subcore_axis_name="s")


def _deg_body(d_ref, z16_ref, ones_ref, deg2_ref, di_v, di_t, ones_v, acc):
    c = lax.axis_index("c")
    s = lax.axis_index("s")
    row0 = s * RPT
    pltpu.sync_copy(z16_ref, acc.at[pl.ds(row0, RPT)])
    pltpu.sync_copy(ones_ref, ones_v)
    plsc.subcore_barrier()

    base0 = (c * NS + s) * DEPT

    @pl.loop(0, DNFULL)
    def _(i):
        b = base0 + i * CHUNK
        pltpu.sync_copy(d_ref.at[pl.ds(b, CHUNK)], di_v)
        pltpu.sync_copy(ones_v, acc.at[di_v], add=True)

    bt = base0 + DNFULL * CHUNK
    pltpu.sync_copy(d_ref.at[pl.ds(bt, DTAIL)], di_t)
    pltpu.sync_copy(ones_v.at[pl.ds(0, DTAIL)], acc.at[di_t], add=True)

    plsc.subcore_barrier()
    pltpu.sync_copy(acc.at[pl.ds(row0, RPT)], deg2_ref.at[c, pl.ds(row0, RPT)])


def _deg_pass(dst):
    z16 = jnp.zeros((RPT, 16), jnp.float32)
    ones16 = jnp.ones((CHUNK, 16), jnp.float32)
    run = pl.kernel(
        _deg_body,
        out_type=jax.ShapeDtypeStruct((NC, N_PAD, 16), jnp.float32),
        mesh=_MESH,
        compiler_params=pltpu.CompilerParams(use_tc_tiling_on_sc=False),
        scratch_types=[
            pltpu.VMEM((CHUNK,), jnp.int32),
            pltpu.VMEM((DTAIL,), jnp.int32),
            pltpu.VMEM((CHUNK, 16), jnp.float32),
            pltpu.VMEM_SHARED((N_PAD, 16), jnp.float32),
        ],
    )
    return run(dst, z16, ones16)


def _edge_body(y_ref, g_ref, d_ref, z32_ref, agg_ref,
               gi_v, di_v, rows_v, gi_t, di_t, rows_t, acc):
    c = lax.axis_index("c")
    s = lax.axis_index("s")
    row0 = s * RPT
    pltpu.sync_copy(z32_ref, acc.at[pl.ds(row0, RPT)])
    plsc.subcore_barrier()

    off = c * jnp.int32(R * N)
    base0 = s * EPT

    @pl.loop(0, NFULL)
    def _(i):
        b = base0 + i * CHUNK
        pltpu.sync_copy(g_ref.at[pl.ds(b, CHUNK)], gi_v)
        pltpu.sync_copy(d_ref.at[pl.ds(b, CHUNK)], di_v)
        for j in range(CHUNK // 16):
            sl = pl.ds(j * 16, 16)
            gi_v[sl] = gi_v[sl] + off
        pltpu.sync_copy(y_ref.at[gi_v], rows_v)
        pltpu.sync_copy(rows_v, acc.at[di_v], add=True)

    bt = base0 + NFULL * CHUNK
    pltpu.sync_copy(g_ref.at[pl.ds(bt, TAIL)], gi_t)
    pltpu.sync_copy(d_ref.at[pl.ds(bt, TAIL)], di_t)
    for j in range(TAIL // 16):
        sl = pl.ds(j * 16, 16)
        gi_t[sl] = gi_t[sl] + off
    pltpu.sync_copy(y_ref.at[gi_t], rows_t)
    pltpu.sync_copy(rows_t, acc.at[di_t], add=True)

    plsc.subcore_barrier()
    pltpu.sync_copy(acc.at[pl.ds(row0, RPT)], agg_ref.at[c, pl.ds(row0, RPT)])


def _edge_pass(y2, g, dst):
    z32 = jnp.zeros((RPT, H), jnp.float32)
    run = pl.kernel(
        _edge_body,
        out_type=jax.ShapeDtypeStruct((NC, N_PAD, H), jnp.float32),
        mesh=_MESH,
        compiler_params=pltpu.CompilerParams(use_tc_tiling_on_sc=False),
        scratch_types=[
            pltpu.VMEM((CHUNK,), jnp.int32),
            pltpu.VMEM((CHUNK,), jnp.int32),
            pltpu.VMEM((CHUNK, H), jnp.float32),
            pltpu.VMEM((TAIL,), jnp.int32),
            pltpu.VMEM((TAIL,), jnp.int32),
            pltpu.VMEM((TAIL, H), jnp.float32),
            pltpu.VMEM_SHARED((N_PAD, H), jnp.float32),
        ],
    )
    return run(y2, g, dst, z32)


BN = 2000


def _wmix_body(coeffs_ref, bases_ref, w_ref):
    w_ref[...] = jnp.dot(coeffs_ref[...], bases_ref[...],
                         preferred_element_type=jnp.float32)


def _wmix(coeffs, bases):
    w = pl.pallas_call(
        _wmix_body,
        out_shape=jax.ShapeDtypeStruct((R, D * D), jnp.float32),
    )(coeffs, bases.reshape(B, D * D))
    return w.reshape(R, D, D)


def _y_body(x_ref, w_ref, y_ref):
    y_ref[0, 0] = jnp.dot(x_ref[...], w_ref[0, 0],
                          preferred_element_type=jnp.float32)


def _y_transform(x, w):
    wsplit = w.reshape(R, D, NC, H).transpose(2, 0, 1, 3)
    return pl.pallas_call(
        _y_body,
        grid=(NC, R, N // BN),
        in_specs=[
            pl.BlockSpec((BN, D), lambda h, r, n: (n, 0)),
            pl.BlockSpec((1, 1, D, H), lambda h, r, n: (h, r, 0, 0)),
        ],
        out_specs=pl.BlockSpec((1, 1, BN, H), lambda h, r, n: (h, r, n, 0)),
        out_shape=jax.ShapeDtypeStruct((NC, R, N, H), jnp.float32),
    )(x, wsplit)


def _self_body(x_ref, w_ref, o_ref):
    o_ref[...] = jnp.dot(x_ref[...], w_ref[...],
                         preferred_element_type=jnp.float32)


def _self_mm(x, w):
    return pl.pallas_call(
        _self_body,
        grid=(N // BN,),
        in_specs=[
            pl.BlockSpec((BN, D), lambda n: (n, 0)),
            pl.BlockSpec((D, D), lambda n: (0, 0)),
        ],
        out_specs=pl.BlockSpec((BN, D), lambda n: (n, 0)),
        out_shape=jax.ShapeDtypeStruct((N, D), jnp.float32),
    )(x, w)


def _finalize_body(relu, agg_ref, deg_ref, self_ref, o_ref):
    deg = deg_ref[0, :, 0] + deg_ref[1, :, 0]
    dinv = (1.0 / jnp.maximum(deg, 1.0))[:, None]
    res = jnp.concatenate([agg_ref[0], agg_ref[1]], axis=1) * dinv + self_ref[...]
    if relu:
        res = jnp.maximum(res, 0.0)
    o_ref[...] = res


def _finalize(agg2, deg2, selfout, relu):
    return pl.pallas_call(
        functools.partial(_finalize_body, relu),
        grid=(N // BN,),
        in_specs=[
            pl.BlockSpec((NC, BN, H), lambda n: (0, n, 0)),
            pl.BlockSpec((NC, BN, 16), lambda n: (0, n, 0)),
            pl.BlockSpec((BN, D), lambda n: (n, 0)),
        ],
        out_specs=pl.BlockSpec((BN, D), lambda n: (n, 0)),
        out_shape=jax.ShapeDtypeStruct((N, D), jnp.float32),
    )(agg2, deg2, selfout)


def _layer(x, bases, coeffs, self_loop, g, dst, deg2, relu):
    w = _wmix(coeffs, bases)
    y = _y_transform(x, w)
    y2 = y.reshape(NC * R * N, H)
    agg2 = _edge_pass(y2, g, dst)
    selfout = _self_mm(x, self_loop)
    return _finalize(agg2, deg2, selfout, relu)


def kernel(emb, bases0, coeffs0, self_loop0, bases1, coeffs1, self_loop1,
           edge_index, edge_type):
    src = edge_index[0]
    dst = edge_index[1]
    g = edge_type * jnp.int32(N) + src
    deg2 = _deg_pass(dst)
    x1 = _layer(emb, bases0, coeffs0, self_loop0, g, dst, deg2, relu=True)
    x2 = _layer(x1, bases1, coeffs1, self_loop1, g, dst, deg2, relu=False)
    return x2


# speedup vs baseline: 15.3463x; 1.6799x over previous
import functools

import jax
import jax.numpy as jnp
from jax import lax
from jax.experimental import pallas as pl
from jax.experimental.pallas import tpu as pltpu
from jax.experimental.pallas import tpu_sc as plsc

N = 50000
E = 800000
D = 64
R = 4
B = 30
H = D // 2

NC = 2
NS = 16
CHUNK = 128
RPT = 3128
N_PAD = RPT * NS

EPT = E // NS
ECHUNK = 80
ECPT = EPT // ECHUNK
NBUF = 5
EGROUPS = ECPT // NBUF

DEPT = (E // NC) // NS
DNFULL = DEPT // CHUNK
DTAIL = DEPT - DNFULL * CHUNK

_MESH = plsc.VectorSubcoreMesh(core_axis_name="c", subcore_axis_name="s")


def _deg_body(d_ref, z16_ref, ones_ref, deg2_ref, di_v, di_t, ones_v, acc):
    c = lax.axis_index("c")
    s = lax.axis_index("s")
    row0 = s * RPT
    pltpu.sync_copy(z16_ref, acc.at[pl.ds(row0, RPT)])
    pltpu.sync_copy(ones_ref, ones_v)
    plsc.subcore_barrier()

    base0 = (c * NS + s) * DEPT

    @pl.loop(0, DNFULL)
    def _(i):
        b = base0 + i * CHUNK
        pltpu.sync_copy(d_ref.at[pl.ds(b, CHUNK)], di_v)
        pltpu.sync_copy(ones_v, acc.at[di_v], add=True)

    bt = base0 + DNFULL * CHUNK
    pltpu.sync_copy(d_ref.at[pl.ds(bt, DTAIL)], di_t)
    pltpu.sync_copy(ones_v.at[pl.ds(0, DTAIL)], acc.at[di_t], add=True)

    plsc.subcore_barrier()
    pltpu.sync_copy(acc.at[pl.ds(row0, RPT)], deg2_ref.at[c, pl.ds(row0, RPT)])


def _deg_pass(dst):
    z16 = jnp.zeros((RPT, 16), jnp.float32)
    ones16 = jnp.ones((CHUNK, 16), jnp.float32)
    run = pl.kernel(
        _deg_body,
        out_type=jax.ShapeDtypeStruct((NC, N_PAD, 16), jnp.float32),
        mesh=_MESH,
        compiler_params=pltpu.CompilerParams(use_tc_tiling_on_sc=False),
        scratch_types=[
            pltpu.VMEM((CHUNK,), jnp.int32),
            pltpu.VMEM((DTAIL,), jnp.int32),
            pltpu.VMEM((CHUNK, 16), jnp.float32),
            pltpu.VMEM_SHARED((N_PAD, 16), jnp.float32),
        ],
    )
    return run(dst, z16, ones16)


def _edge_body(y_ref, g2_ref, d_ref, z32_ref, agg_ref,
               gi0, gi1, gi2, gi3, gi4, di0, di1, di2, di3, di4,
               r0, r1, r2, r3, r4, isem, gsem, acc):
    c = lax.axis_index("c")
    s = lax.axis_index("s")
    gis = [gi0, gi1, gi2, gi3, gi4]
    dis = [di0, di1, di2, di3, di4]
    rows = [r0, r1, r2, r3, r4]
    row0 = s * RPT
    pltpu.sync_copy(z32_ref, acc.at[pl.ds(row0, RPT)])

    base0 = s * EPT
    for b in range(NBUF):
        pltpu.async_copy(g2_ref.at[c, pl.ds(base0 + b * ECHUNK, ECHUNK)],
                         gis[b], isem.at[b])
        pltpu.async_copy(d_ref.at[pl.ds(base0 + b * ECHUNK, ECHUNK)],
                         dis[b], isem.at[b])
    plsc.subcore_barrier()

    @pl.loop(0, EGROUPS)
    def _(gidx):
        k0 = base0 + gidx * (NBUF * ECHUNK)
        for b in range(NBUF):
            pltpu.make_async_copy(g2_ref.at[c, pl.ds(base0, ECHUNK)],
                                  gis[b], isem.at[b]).wait()
            pltpu.make_async_copy(d_ref.at[pl.ds(base0, ECHUNK)],
                                  dis[b], isem.at[b]).wait()
            pltpu.async_copy(y_ref.at[gis[b]], rows[b], gsem.at[b])
        for b in range(NBUF):
            pltpu.make_async_copy(y_ref.at[gis[b]], rows[b],
                                  gsem.at[b]).wait()
            pltpu.sync_copy(rows[b], acc.at[dis[b]], add=True)

            @pl.when(gidx < EGROUPS - 1)
            def _():
                nxt = k0 + (NBUF + b) * ECHUNK
                pltpu.async_copy(g2_ref.at[c, pl.ds(nxt, ECHUNK)],
                                 gis[b], isem.at[b])
                pltpu.async_copy(d_ref.at[pl.ds(nxt, ECHUNK)],
                                 dis[b], isem.at[b])

    plsc.subcore_barrier()
    pltpu.sync_copy(acc.at[pl.ds(row0, RPT)], agg_ref.at[c, pl.ds(row0, RPT)])


def _edge_pass(y2, g2, dst):
    z32 = jnp.zeros((RPT, H), jnp.float32)
    run = pl.kernel(
        _edge_body,
        out_type=jax.ShapeDtypeStruct((NC, N_PAD, H), jnp.float32),
        mesh=_MESH,
        compiler_params=pltpu.CompilerParams(use_tc_tiling_on_sc=False),
        scratch_types=(
            [pltpu.VMEM((ECHUNK,), jnp.int32) for _ in range(2 * NBUF)]
            + [pltpu.VMEM((ECHUNK, H), jnp.float32) for _ in range(NBUF)]
            + [pltpu.SemaphoreType.DMA((NBUF,)),
               pltpu.SemaphoreType.DMA((NBUF,)),
               pltpu.VMEM_SHARED((N_PAD, H), jnp.float32)]
        ),
    )
    return run(y2, g2, dst, z32)


BN = 2000


def _wmix_body(coeffs_ref, bases_ref, w_ref):
    w_ref[...] = jnp.dot(coeffs_ref[...], bases_ref[...],
                         preferred_element_type=jnp.float32)


def _wmix(coeffs, bases):
    w = pl.pallas_call(
        _wmix_body,
        out_shape=jax.ShapeDtypeStruct((R, D * D), jnp.float32),
    )(coeffs, bases.reshape(B, D * D))
    return w.reshape(R, D, D)


def _y_body(x_ref, w_ref, y_ref):
    y_ref[0, 0] = jnp.dot(x_ref[...], w_ref[0, 0],
                          preferred_element_type=jnp.float32)


def _y_transform(x, w):
    wsplit = w.reshape(R, D, NC, H).transpose(2, 0, 1, 3)
    return pl.pallas_call(
        _y_body,
        grid=(NC, R, N // BN),
        in_specs=[
            pl.BlockSpec((BN, D), lambda h, r, n: (n, 0)),
            pl.BlockSpec((1, 1, D, H), lambda h, r, n: (h, r, 0, 0)),
        ],
        out_specs=pl.BlockSpec((1, 1, BN, H), lambda h, r, n: (h, r, n, 0)),
        out_shape=jax.ShapeDtypeStruct((NC, R, N, H), jnp.float32),
    )(x, wsplit)


def _self_body(x_ref, w_ref, o_ref):
    o_ref[...] = jnp.dot(x_ref[...], w_ref[...],
                         preferred_element_type=jnp.float32)


def _self_mm(x, w):
    return pl.pallas_call(
        _self_body,
        grid=(N // BN,),
        in_specs=[
            pl.BlockSpec((BN, D), lambda n: (n, 0)),
            pl.BlockSpec((D, D), lambda n: (0, 0)),
        ],
        out_specs=pl.BlockSpec((BN, D), lambda n: (n, 0)),
        out_shape=jax.ShapeDtypeStruct((N, D), jnp.float32),
    )(x, w)


def _finalize_body(relu, agg_ref, deg_ref, self_ref, o_ref):
    deg = deg_ref[0, :, 0] + deg_ref[1, :, 0]
    dinv = (1.0 / jnp.maximum(deg, 1.0))[:, None]
    res = jnp.concatenate([agg_ref[0], agg_ref[1]], axis=1) * dinv + self_ref[...]
    if relu:
        res = jnp.maximum(res, 0.0)
    o_ref[...] = res


def _finalize(agg2, deg2, selfout, relu):
    return pl.pallas_call(
        functools.partial(_finalize_body, relu),
        grid=(N // BN,),
        in_specs=[
            pl.BlockSpec((NC, BN, H), lambda n: (0, n, 0)),
            pl.BlockSpec((NC, BN, 16), lambda n: (0, n, 0)),
            pl.BlockSpec((BN, D), lambda n: (n, 0)),
        ],
        out_specs=pl.BlockSpec((BN, D), lambda n: (n, 0)),
        out_shape=jax.ShapeDtypeStruct((N, D), jnp.float32),
    )(agg2, deg2, selfout)


def _layer(x, bases, coeffs, self_loop, g2, d2, deg2, relu):
    w = _wmix(coeffs, bases)
    y = _y_transform(x, w)
    y2 = y.reshape(NC * R * N, H)
    agg2 = _edge_pass(y2, g2, d2)
    selfout = _self_mm(x, self_loop)
    return _finalize(agg2, deg2, selfout, relu)


def kernel(emb, bases0, coeffs0, self_loop0, bases1, coeffs1, self_loop1,
           edge_index, edge_type):
    src = edge_index[0]
    dst = edge_index[1]
    g = edge_type * jnp.int32(N) + src
    g2 = g[None, :] + jnp.int32(R * N) * jnp.arange(NC, dtype=jnp.int32)[:, None]
    deg2 = _deg_pass(dst)
    x1 = _layer(emb, bases0, coeffs0, self_loop0, g2, dst, deg2, relu=True)
    x2 = _layer(x1, bases1, coeffs1, self_loop1, g2, dst, deg2, relu=False)
    return x2


# speedup vs baseline: 18.8871x; 1.2307x over previous
import functools

import jax
import jax.numpy as jnp
from jax import lax
from jax.experimental import pallas as pl
from jax.experimental.pallas import tpu as pltpu
from jax.experimental.pallas import tpu_sc as plsc

N = 50000
E = 800000
D = 64
R = 4
B = 30
H = D // 2

NC = 2
NS = 16
CHUNK = 128
RPT = 3128
N_PAD = RPT * NS

EPT = E // NS
ECHUNK = 80
ECPT = EPT // ECHUNK
NBUF = 5
EGROUPS = ECPT // NBUF

DEPT = (E // NC) // NS
DNFULL = DEPT // CHUNK
DTAIL = DEPT - DNFULL * CHUNK

_MESH = plsc.VectorSubcoreMesh(core_axis_name="c", subcore_axis_name="s")


def _deg_body(d_ref, z16_ref, ones_ref, deg2_ref, di_v, di_t, ones_v, acc):
    c = lax.axis_index("c")
    s = lax.axis_index("s")
    row0 = s * RPT
    pltpu.sync_copy(z16_ref, acc.at[pl.ds(row0, RPT)])
    pltpu.sync_copy(ones_ref, ones_v)
    plsc.subcore_barrier()

    base0 = (c * NS + s) * DEPT

    @pl.loop(0, DNFULL)
    def _(i):
        b = base0 + i * CHUNK
        pltpu.sync_copy(d_ref.at[pl.ds(b, CHUNK)], di_v)
        pltpu.sync_copy(ones_v, acc.at[di_v], add=True)

    bt = base0 + DNFULL * CHUNK
    pltpu.sync_copy(d_ref.at[pl.ds(bt, DTAIL)], di_t)
    pltpu.sync_copy(ones_v.at[pl.ds(0, DTAIL)], acc.at[di_t], add=True)

    plsc.subcore_barrier()
    pltpu.sync_copy(acc.at[pl.ds(row0, RPT)], deg2_ref.at[c, pl.ds(row0, RPT)])


def _deg_pass(dst):
    z16 = jnp.zeros((RPT, 16), jnp.float32)
    ones16 = jnp.ones((CHUNK, 16), jnp.float32)
    run = pl.kernel(
        _deg_body,
        out_type=jax.ShapeDtypeStruct((NC, N_PAD, 16), jnp.float32),
        mesh=_MESH,
        compiler_params=pltpu.CompilerParams(use_tc_tiling_on_sc=False),
        scratch_types=[
            pltpu.VMEM((CHUNK,), jnp.int32),
            pltpu.VMEM((DTAIL,), jnp.int32),
            pltpu.VMEM((CHUNK, 16), jnp.float32),
            pltpu.VMEM_SHARED((N_PAD, 16), jnp.float32),
        ],
    )
    return run(dst, z16, ones16)


def _edge_body(y_ref, g2_ref, d_ref, z32_ref, agg_ref,
               gi0, gi1, gi2, gi3, gi4, di0, di1, di2, di3, di4,
               r0, r1, r2, r3, r4, isem, gsem, acc):
    c = lax.axis_index("c")
    s = lax.axis_index("s")
    gis = [gi0, gi1, gi2, gi3, gi4]
    dis = [di0, di1, di2, di3, di4]
    rows = [r0, r1, r2, r3, r4]
    row0 = s * RPT
    pltpu.sync_copy(z32_ref, acc.at[pl.ds(row0, RPT)])

    base0 = s * EPT
    for b in range(NBUF):
        pltpu.async_copy(g2_ref.at[c, pl.ds(base0 + b * ECHUNK, ECHUNK)],
                         gis[b], isem.at[b])
        pltpu.async_copy(d_ref.at[pl.ds(base0 + b * ECHUNK, ECHUNK)],
                         dis[b], isem.at[b])
    plsc.subcore_barrier()

    @pl.loop(0, EGROUPS)
    def _(gidx):
        k0 = base0 + gidx * (NBUF * ECHUNK)
        for b in range(NBUF):
            pltpu.make_async_copy(g2_ref.at[c, pl.ds(base0, ECHUNK)],
                                  gis[b], isem.at[b]).wait()
            pltpu.make_async_copy(d_ref.at[pl.ds(base0, ECHUNK)],
                                  dis[b], isem.at[b]).wait()
            pltpu.async_copy(y_ref.at[gis[b]], rows[b], gsem.at[b])
        for b in range(NBUF):
            pltpu.make_async_copy(y_ref.at[gis[b]], rows[b],
                                  gsem.at[b]).wait()
            pltpu.sync_copy(rows[b], acc.at[dis[b]], add=True)

            @pl.when(gidx < EGROUPS - 1)
            def _():
                nxt = k0 + (NBUF + b) * ECHUNK
                pltpu.async_copy(g2_ref.at[c, pl.ds(nxt, ECHUNK)],
                                 gis[b], isem.at[b])
                pltpu.async_copy(d_ref.at[pl.ds(nxt, ECHUNK)],
                                 dis[b], isem.at[b])

    plsc.subcore_barrier()
    pltpu.sync_copy(acc.at[pl.ds(row0, RPT)], agg_ref.at[c, pl.ds(row0, RPT)])


def _edge_pass(y2, g2, dst):
    z32 = jnp.zeros((RPT, H), jnp.float32)
    run = pl.kernel(
        _edge_body,
        out_type=jax.ShapeDtypeStruct((NC, N_PAD, H), jnp.float32),
        mesh=_MESH,
        compiler_params=pltpu.CompilerParams(use_tc_tiling_on_sc=False),
        scratch_types=(
            [pltpu.VMEM((ECHUNK,), jnp.int32) for _ in range(2 * NBUF)]
            + [pltpu.VMEM((ECHUNK, H), jnp.float32) for _ in range(NBUF)]
            + [pltpu.SemaphoreType.DMA((NBUF,)),
               pltpu.SemaphoreType.DMA((NBUF,)),
               pltpu.VMEM_SHARED((N_PAD, H), jnp.float32)]
        ),
    )
    return run(y2, g2, dst, z32)


BN = 2000


def _wmix_body(coeffs_ref, bases_ref, w_ref):
    w_ref[...] = jnp.dot(coeffs_ref[...], bases_ref[...],
                         preferred_element_type=jnp.float32)


def _wmix(coeffs, bases):
    w = pl.pallas_call(
        _wmix_body,
        out_shape=jax.ShapeDtypeStruct((R, D * D), jnp.float32),
    )(coeffs, bases.reshape(B, D * D))
    return w.reshape(R, D, NC, H).transpose(2, 0, 1, 3)


def _y_body(w_ref, x_ref, y_ref):
    x = x_ref[...]
    for h in range(NC):
        for r in range(R):
            y_ref[h, r] = jnp.dot(x, w_ref[h, r],
                                  preferred_element_type=jnp.float32)


def _y_transform(wsplit, x):
    return pl.pallas_call(
        _y_body,
        grid=(N // BN,),
        in_specs=[
            pl.BlockSpec((NC, R, D, H), lambda n: (0, 0, 0, 0)),
            pl.BlockSpec((BN, D), lambda n: (n, 0)),
        ],
        out_specs=pl.BlockSpec((NC, R, BN, H), lambda n: (0, 0, n, 0)),
        out_shape=jax.ShapeDtypeStruct((NC, R, N, H), jnp.float32),
    )(wsplit, x)


def _finalize_body(relu, agg_ref, deg_ref, x_ref, sw_ref, o_ref):
    deg = deg_ref[0, :, 0] + deg_ref[1, :, 0]
    dinv = (1.0 / jnp.maximum(deg, 1.0))[:, None]
    res = (jnp.concatenate([agg_ref[0], agg_ref[1]], axis=1) * dinv
           + jnp.dot(x_ref[...], sw_ref[...],
                     preferred_element_type=jnp.float32))
    if relu:
        res = jnp.maximum(res, 0.0)
    o_ref[...] = res


def _finalize(agg2, deg2, x, self_loop, relu):
    return pl.pallas_call(
        functools.partial(_finalize_body, relu),
        grid=(N // BN,),
        in_specs=[
            pl.BlockSpec((NC, BN, H), lambda n: (0, n, 0)),
            pl.BlockSpec((NC, BN, 16), lambda n: (0, n, 0)),
            pl.BlockSpec((BN, D), lambda n: (n, 0)),
            pl.BlockSpec((D, D), lambda n: (0, 0)),
        ],
        out_specs=pl.BlockSpec((BN, D), lambda n: (n, 0)),
        out_shape=jax.ShapeDtypeStruct((N, D), jnp.float32),
    )(agg2, deg2, x, self_loop)


def _layer(x, bases, coeffs, self_loop, g2, d2, deg2, relu):
    wsplit = _wmix(coeffs, bases)
    y = _y_transform(wsplit, x)
    y2 = y.reshape(NC * R * N, H)
    agg2 = _edge_pass(y2, g2, d2)
    return _finalize(agg2, deg2, x, self_loop, relu)


def kernel(emb, bases0, coeffs0, self_loop0, bases1, coeffs1, self_loop1,
           edge_index, edge_type):
    src = edge_index[0]
    dst = edge_index[1]
    g = edge_type * jnp.int32(N) + src
    g2 = g[None, :] + jnp.int32(R * N) * jnp.arange(NC, dtype=jnp.int32)[:, None]
    deg2 = _deg_pass(dst)
    x1 = _layer(emb, bases0, coeffs0, self_loop0, g2, dst, deg2, relu=True)
    x2 = _layer(x1, bases1, coeffs1, self_loop1, g2, dst, deg2, relu=False)
    return x2


# speedup vs baseline: 20.0405x; 1.0611x over previous
import functools

import jax
import jax.numpy as jnp
from jax import lax
from jax.experimental import pallas as pl
from jax.experimental.pallas import tpu as pltpu
from jax.experimental.pallas import tpu_sc as plsc

N = 50000
E = 800000
D = 64
R = 4
B = 30
H = D // 2

NC = 2
NS = 16
CHUNK = 128
RPT = 3128
N_PAD = RPT * NS

EPT = E // NS
ECHUNK = 80
ECPT = EPT // ECHUNK
NBUF = 5
EGROUPS = ECPT // NBUF

DEPT = (E // NC) // NS
DCH = 80
DNB = 4
DGROUPS = 78
DTAIL = DEPT - DGROUPS * DNB * DCH

_MESH = plsc.VectorSubcoreMesh(core_axis_name="c", subcore_axis_name="s")


def _deg_body(d_ref, z16_ref, ones_ref, deg2_ref,
              di0, di1, di2, di3, di_t, ones_v, isem, acc):
    c = lax.axis_index("c")
    s = lax.axis_index("s")
    dis = [di0, di1, di2, di3]
    row0 = s * RPT
    pltpu.sync_copy(z16_ref, acc.at[pl.ds(row0, RPT)])
    pltpu.sync_copy(ones_ref, ones_v)

    base0 = (c * NS + s) * DEPT
    for b in range(DNB):
        pltpu.async_copy(d_ref.at[pl.ds(base0 + b * DCH, DCH)],
                         dis[b], isem.at[b])
    plsc.subcore_barrier()

    @pl.loop(0, DGROUPS)
    def _(gidx):
        k0 = base0 + gidx * (DNB * DCH)
        for b in range(DNB):
            pltpu.make_async_copy(d_ref.at[pl.ds(base0, DCH)],
                                  dis[b], isem.at[b]).wait()
            pltpu.sync_copy(ones_v, acc.at[dis[b]], add=True)

            @pl.when(gidx < DGROUPS - 1)
            def _():
                pltpu.async_copy(d_ref.at[pl.ds(k0 + (DNB + b) * DCH, DCH)],
                                 dis[b], isem.at[b])

    bt = base0 + DGROUPS * DNB * DCH
    pltpu.sync_copy(d_ref.at[pl.ds(bt, DTAIL)], di_t)
    pltpu.sync_copy(ones_v.at[pl.ds(0, DTAIL)], acc.at[di_t], add=True)

    plsc.subcore_barrier()
    pltpu.sync_copy(acc.at[pl.ds(row0, RPT)], deg2_ref.at[c, pl.ds(row0, RPT)])


def _deg_pass(dst):
    z16 = jnp.zeros((RPT, 16), jnp.float32)
    ones16 = jnp.ones((DCH, 16), jnp.float32)
    run = pl.kernel(
        _deg_body,
        out_type=jax.ShapeDtypeStruct((NC, N_PAD, 16), jnp.float32),
        mesh=_MESH,
        compiler_params=pltpu.CompilerParams(use_tc_tiling_on_sc=False),
        scratch_types=(
            [pltpu.VMEM((DCH,), jnp.int32) for _ in range(DNB)]
            + [pltpu.VMEM((DTAIL,), jnp.int32),
               pltpu.VMEM((DCH, 16), jnp.float32),
               pltpu.SemaphoreType.DMA((DNB,)),
               pltpu.VMEM_SHARED((N_PAD, 16), jnp.float32)]
        ),
    )
    return run(dst, z16, ones16)


def _edge_body(y_ref, g2_ref, d_ref, z32_ref, agg_ref,
               gi0, gi1, gi2, gi3, gi4, di0, di1, di2, di3, di4,
               r0, r1, r2, r3, r4, isem, gsem, acc):
    c = lax.axis_index("c")
    s = lax.axis_index("s")
    gis = [gi0, gi1, gi2, gi3, gi4]
    dis = [di0, di1, di2, di3, di4]
    rows = [r0, r1, r2, r3, r4]
    row0 = s * RPT
    pltpu.sync_copy(z32_ref, acc.at[pl.ds(row0, RPT)])

    base0 = s * EPT
    for b in range(NBUF):
        pltpu.async_copy(g2_ref.at[c, pl.ds(base0 + b * ECHUNK, ECHUNK)],
                         gis[b], isem.at[b])
        pltpu.async_copy(d_ref.at[pl.ds(base0 + b * ECHUNK, ECHUNK)],
                         dis[b], isem.at[b])
    plsc.subcore_barrier()

    @pl.loop(0, EGROUPS)
    def _(gidx):
        k0 = base0 + gidx * (NBUF * ECHUNK)
        for b in range(NBUF):
            pltpu.make_async_copy(g2_ref.at[c, pl.ds(base0, ECHUNK)],
                                  gis[b], isem.at[b]).wait()
            pltpu.make_async_copy(d_ref.at[pl.ds(base0, ECHUNK)],
                                  dis[b], isem.at[b]).wait()

            pltpu.async_copy(y_ref.at[gis[b]], rows[b], gsem.at[b])
        for b in range(NBUF):
            pltpu.make_async_copy(y_ref.at[gis[b]], rows[b],
                                  gsem.at[b]).wait()
            pltpu.sync_copy(rows[b], acc.at[dis[b]], add=True)

            @pl.when(gidx < EGROUPS - 1)
            def _():
                nxt = k0 + (NBUF + b) * ECHUNK
                pltpu.async_copy(g2_ref.at[c, pl.ds(nxt, ECHUNK)],
                                 gis[b], isem.at[b])
                pltpu.async_copy(d_ref.at[pl.ds(nxt, ECHUNK)],
                                 dis[b], isem.at[b])

    plsc.subcore_barrier()
    pltpu.sync_copy(acc.at[pl.ds(row0, RPT)], agg_ref.at[c, pl.ds(row0, RPT)])


def _edge_pass(y2, g2, dst):
    z32 = jnp.zeros((RPT, H), jnp.float32)
    run = pl.kernel(
        _edge_body,
        out_type=jax.ShapeDtypeStruct((NC, N_PAD, H), jnp.float32),
        mesh=_MESH,
        compiler_params=pltpu.CompilerParams(use_tc_tiling_on_sc=False),
        scratch_types=(
            [pltpu.VMEM((ECHUNK,), jnp.int32) for _ in range(2 * NBUF)]
            + [pltpu.VMEM((ECHUNK, H), jnp.float32) for _ in range(NBUF)]
            + [pltpu.SemaphoreType.DMA((NBUF,)),
               pltpu.SemaphoreType.DMA((NBUF,)),
               pltpu.VMEM_SHARED((N_PAD, H), jnp.float32)]
        ),
    )
    return run(y2, g2, dst, z32)


BN = 2000


def _wmix_body(coeffs_ref, bases_ref, w_ref):
    w_ref[...] = jnp.dot(coeffs_ref[...], bases_ref[...],
                         preferred_element_type=jnp.float32)


def _wmix(coeffs, bases):
    w = pl.pallas_call(
        _wmix_body,
        out_shape=jax.ShapeDtypeStruct((R, D * D), jnp.float32),
    )(coeffs, bases.reshape(B, D * D))
    return w.reshape(R, D, NC, H).transpose(2, 0, 1, 3)


def _y_body(w_ref, x_ref, y_ref):
    x = x_ref[...]
    for h in range(NC):
        for r in range(R):
            y_ref[h, r] = jnp.dot(x, w_ref[h, r],
                                  preferred_element_type=jnp.float32)


def _y_transform(wsplit, x):
    return pl.pallas_call(
        _y_body,
        grid=(N // BN,),
        in_specs=[
            pl.BlockSpec((NC, R, D, H), lambda n: (0, 0, 0, 0)),
            pl.BlockSpec((BN, D), lambda n: (n, 0)),
        ],
        out_specs=pl.BlockSpec((NC, R, BN, H), lambda n: (0, 0, n, 0)),
        out_shape=jax.ShapeDtypeStruct((NC, R, N, H), jnp.float32),
    )(wsplit, x)


def _finalize_body(relu, agg_ref, deg_ref, x_ref, sw_ref, o_ref):
    deg = deg_ref[0, :, 0] + deg_ref[1, :, 0]
    dinv = (1.0 / jnp.maximum(deg, 1.0))[:, None]
    res = (jnp.concatenate([agg_ref[0], agg_ref[1]], axis=1) * dinv
           + jnp.dot(x_ref[...], sw_ref[...],
                     preferred_element_type=jnp.float32))
    if relu:
        res = jnp.maximum(res, 0.0)
    o_ref[...] = res


def _finalize(agg2, deg2, x, self_loop, relu):
    return pl.pallas_call(
        functools.partial(_finalize_body, relu),
        grid=(N // BN,),
        in_specs=[
            pl.BlockSpec((NC, BN, H), lambda n: (0, n, 0)),
            pl.BlockSpec((NC, BN, 16), lambda n: (0, n, 0)),
            pl.BlockSpec((BN, D), lambda n: (n, 0)),
            pl.BlockSpec((D, D), lambda n: (0, 0)),
        ],
        out_specs=pl.BlockSpec((BN, D), lambda n: (n, 0)),
        out_shape=jax.ShapeDtypeStruct((N, D), jnp.float32),
    )(agg2, deg2, x, self_loop)


def _layer(x, bases, coeffs, self_loop, g2, d2, deg2, relu):
    wsplit = _wmix(coeffs, bases)
    y = _y_transform(wsplit, x)
    y2 = y.reshape(NC * R * N, H)
    agg2 = _edge_pass(y2, g2, d2)
    return _finalize(agg2, deg2, x, self_loop, relu)


def kernel(emb, bases0, coeffs0, self_loop0, bases1, coeffs1, self_loop1,
           edge_index, edge_type):
    src = edge_index[0]
    dst = edge_index[1]
    g = edge_type * jnp.int32(N) + src
    g2 = g[None, :] + jnp.int32(R * N) * jnp.arange(NC, dtype=jnp.int32)[:, None]
    deg2 = _deg_pass(dst)
    x1 = _layer(emb, bases0, coeffs0, self_loop0, g2, dst, deg2, relu=True)
    x2 = _layer(x1, bases1, coeffs1, self_loop1, g2, dst, deg2, relu=False)
    return x2
